# Initial kernel scaffold; baseline (speedup 1.0000x reference)
import jax
import jax.numpy as jnp
from jax import lax
from jax.experimental import pallas as pl
from jax.experimental.pallas import tpu as pltpu
from jax.experimental.pallas import tpu_sc as plsc

N = 10000
E = 160000
G = 64
IN_F = 256
HID_F = 512
OUT_F = 512

NC = 2
NS = 16
NW = NC * NS
L = 16

EPT = E // NW
EPT_PAD = 5008
DEG_PAD = N + L


def _deg_body(dst_hbm, out_hbm, dst_v, deg_v, ones_v):
    c = lax.axis_index("c")
    s = lax.axis_index("s")
    wid = s * NC + c
    pltpu.sync_copy(dst_hbm.at[wid], dst_v)

    @pl.loop(0, DEG_PAD // L)
    def _zero(i):
        deg_v[pl.ds(i * L, L)] = jnp.zeros((L,), jnp.float32)

    ones_v[...] = jnp.ones((L,), jnp.float32)

    @pl.loop(0, EPT_PAD // L)
    def _scat(j):
        idx = dst_v[pl.ds(j * L, L)]
        plsc.addupdate_scatter(deg_v, [idx], ones_v[...])

    pltpu.sync_copy(deg_v.at[pl.ds(0, N)], out_hbm.at[wid])


def _deg_call(dst_pad):
    mesh = plsc.VectorSubcoreMesh(core_axis_name="c", subcore_axis_name="s")
    return pl.kernel(
        _deg_body,
        out_type=jax.ShapeDtypeStruct((NW, N), jnp.float32),
        mesh=mesh,
        scratch_types=[
            pltpu.VMEM((EPT_PAD,), jnp.int32),
            pltpu.VMEM((DEG_PAD,), jnp.float32),
            pltpu.VMEM((L,), jnp.float32),
        ],
    )(dst_pad)


RB = 1000
NBLK = N // RB


def _prescale_body(parts_ref, x_ref, xsa_ref, xsb_ref):
    deg = jnp.sum(parts_ref[...], axis=0) + 1.0
    dinv = lax.rsqrt(deg)
    xs = x_ref[...] * dinv[:, None]
    xsa_ref[...] = xs[:, :128]
    xsb_ref[...] = xs[:, 128:]


def _prescale_call(deg_parts, x):
    return pl.pallas_call(
        _prescale_body,
        grid=(NBLK,),
        in_specs=[
            pl.BlockSpec((NW, RB), lambda i: (0, i)),
            pl.BlockSpec((RB, IN_F), lambda i: (i, 0)),
        ],
        out_specs=[
            pl.BlockSpec((RB, 128), lambda i: (i, 0)),
            pl.BlockSpec((RB, 128), lambda i: (i, 0)),
        ],
        out_shape=[
            jax.ShapeDtypeStruct((N, 128), jnp.float32),
            jax.ShapeDtypeStruct((N, 128), jnp.float32),
        ],
    )(deg_parts, x)


EK = 80
NCHUNK = (E // NS) // EK
RPT = N // NS
ZR = 125


def _agg_body(src_hbm, dst_hbm, xsa_hbm, xsb_hbm, ax_hbm,
              acc, zbuf, sidx, didx, buf, sem):
    c = lax.axis_index("c")
    s = lax.axis_index("s")

    pltpu.sync_copy(src_hbm.at[s], sidx)
    pltpu.sync_copy(dst_hbm.at[s], didx)

    @pl.loop(0, ZR)
    def _zrow(r):
        for k in range(128 // L):
            zbuf[r, pl.ds(k * L, L)] = jnp.zeros((L,), jnp.float32)

    @pl.loop(0, RPT // ZR)
    def _zacc(b):
        pltpu.sync_copy(zbuf, acc.at[pl.ds(s * RPT + b * ZR, ZR)])

    plsc.subcore_barrier()

    def run(xs_hbm):
        @pl.loop(0, NCHUNK)
        def _edge(j):
            pltpu.async_copy(xs_hbm.at[sidx.at[j]], buf, sem).wait()
            pltpu.sync_copy(buf, acc.at[didx.at[j]], add=True)

    @pl.when(c == 0)
    def _():
        run(xsa_hbm)

    @pl.when(c == 1)
    def _():
        run(xsb_hbm)

    plsc.subcore_barrier()
    pltpu.sync_copy(acc.at[pl.ds(s * RPT, RPT)],
                    ax_hbm.at[c, pl.ds(s * RPT, RPT)])


def _agg_call(src_r, dst_r, xs_a, xs_b):
    mesh = plsc.VectorSubcoreMesh(core_axis_name="c", subcore_axis_name="s")
    return pl.kernel(
        _agg_body,
        out_type=jax.ShapeDtypeStruct((NC, N, 128), jnp.float32),
        mesh=mesh,
        scratch_types=[
            pltpu.VMEM_SHARED((N, 128), jnp.float32),
            pltpu.VMEM((ZR, 128), jnp.float32),
            pltpu.VMEM((NCHUNK, EK), jnp.int32),
            pltpu.VMEM((NCHUNK, EK), jnp.int32),
            pltpu.VMEM((EK, 128), jnp.float32),
            pltpu.SemaphoreType.DMA,
        ],
    )(src_r, dst_r, xs_a, xs_b)


def _tail_body(parts_ref, axa_ref, axb_ref, xsa_ref, xsb_ref,
               batch_blk_ref, batch_full_ref,
               W1_ref, b1_ref, W2_ref, b2_ref,
               node_ref, graph_ref, pool_acc):
    i = pl.program_id(0)
    deg = jnp.sum(parts_ref[...], axis=0) + 1.0
    dinv = lax.rsqrt(deg)
    agg_a = (axa_ref[...] + xsa_ref[...]) * dinv[:, None]
    agg_b = (axb_ref[...] + xsb_ref[...]) * dinv[:, None]
    agg = jnp.concatenate([agg_a, agg_b], axis=1)
    h = jnp.dot(agg, W1_ref[...], preferred_element_type=jnp.float32)
    h = jnp.tanh(h + b1_ref[...])
    node_ref[...] = h

    bblk = batch_blk_ref[0, 0, :]
    gids = lax.broadcasted_iota(jnp.int32, (G, RB), 0)
    P = (bblk[None, :] == gids).astype(jnp.float32)
    part = jnp.dot(P, h, preferred_element_type=jnp.float32)

    @pl.when(i == 0)
    def _():
        pool_acc[...] = part

    @pl.when(i > 0)
    def _():
        pool_acc[...] += part

    @pl.when(i == NBLK - 1)
    def _():
        bf = batch_full_ref[0, :]
        gall = lax.broadcasted_iota(jnp.int32, (G, N), 0)
        cnt = jnp.sum((bf[None, :] == gall).astype(jnp.float32), axis=1)
        mean = pool_acc[...] / jnp.maximum(cnt, 1.0)[:, None]
        g = jnp.dot(mean, W2_ref[...], preferred_element_type=jnp.float32)
        graph_ref[...] = jnp.tanh(g + b2_ref[...])


def _tail_call(deg_parts, ax_a, ax_b, xs_a, xs_b, batch, W1, b1, W2, b2):
    batch_blk = batch.reshape(NBLK, 1, RB)
    batch_full = batch.reshape(1, N)
    return pl.pallas_call(
        _tail_body,
        grid=(NBLK,),
        in_specs=[
            pl.BlockSpec((NW, RB), lambda i: (0, i)),
            pl.BlockSpec((RB, 128), lambda i: (i, 0)),
            pl.BlockSpec((RB, 128), lambda i: (i, 0)),
            pl.BlockSpec((RB, 128), lambda i: (i, 0)),
            pl.BlockSpec((RB, 128), lambda i: (i, 0)),
            pl.BlockSpec((1, 1, RB), lambda i: (i, 0, 0)),
            pl.BlockSpec((1, N), lambda i: (0, 0)),
            pl.BlockSpec((IN_F, HID_F), lambda i: (0, 0)),
            pl.BlockSpec((1, HID_F), lambda i: (0, 0)),
            pl.BlockSpec((HID_F, OUT_F), lambda i: (0, 0)),
            pl.BlockSpec((1, OUT_F), lambda i: (0, 0)),
        ],
        out_specs=[
            pl.BlockSpec((RB, HID_F), lambda i: (i, 0)),
            pl.BlockSpec((G, OUT_F), lambda i: (0, 0)),
        ],
        out_shape=[
            jax.ShapeDtypeStruct((N, HID_F), jnp.float32),
            jax.ShapeDtypeStruct((G, OUT_F), jnp.float32),
        ],
        scratch_shapes=[pltpu.VMEM((G, HID_F), jnp.float32)],
    )(deg_parts, ax_a, ax_b, xs_a, xs_b, batch_blk, batch_full,
      W1, b1.reshape(1, HID_F), W2, b2.reshape(1, OUT_F))


def kernel(x, edge_index, batch, W1, b1, W2, b2):
    src = edge_index[0].astype(jnp.int32)
    dst = edge_index[1].astype(jnp.int32)

    dst_pad = jnp.concatenate(
        [dst.reshape(NW, EPT),
         jnp.full((NW, EPT_PAD - EPT), N, dtype=jnp.int32)], axis=1)
    deg_parts = _deg_call(dst_pad)

    xs_a, xs_b = _prescale_call(deg_parts, x)

    src_r = src.reshape(NS, NCHUNK, EK)
    dst_r = dst.reshape(NS, NCHUNK, EK)
    ax = _agg_call(src_r, dst_r, xs_a, xs_b)

    node, graph = _tail_call(deg_parts, ax[0], ax[1], xs_a, xs_b,
                             batch.astype(jnp.int32), W1, b1, W2, b2)
    return (graph, node)


# speedup vs baseline: 20.4154x; 20.4154x over previous
import jax
import jax.numpy as jnp
from jax import lax
from jax.experimental import pallas as pl
from jax.experimental.pallas import tpu as pltpu
from jax.experimental.pallas import tpu_sc as plsc

N = 10000
E = 160000
G = 64
IN_F = 256
HID_F = 512
OUT_F = 512

NC = 2
NS = 16
NW = NC * NS
L = 16

EPT = E // NW
DK = 100
DCH = EPT // DK
DEG_PAD = 10240
DZR = DEG_PAD // NS


def _deg_body(dst_hbm, out_hbm, deg_acc, didx, ones_v, zbuf):
    c = lax.axis_index("c")
    s = lax.axis_index("s")
    wid = s * NC + c
    pltpu.sync_copy(dst_hbm.at[wid], didx)

    @pl.loop(0, DZR // L)
    def _z(i):
        zbuf[pl.ds(i * L, L)] = jnp.zeros((L,), jnp.float32)

    @pl.loop(0, 112 // L)
    def _o(i):
        ones_v[pl.ds(i * L, L)] = jnp.ones((L,), jnp.float32)

    pltpu.sync_copy(zbuf, deg_acc.at[pl.ds(s * DZR, DZR)])
    plsc.subcore_barrier()

    @pl.loop(0, DCH)
    def _scat(j):
        pltpu.sync_copy(ones_v.at[pl.ds(0, DK)],
                        deg_acc.at[didx.at[j]], add=True)

    plsc.subcore_barrier()
    pltpu.sync_copy(deg_acc.at[pl.ds(s * DZR, DZR)],
                    out_hbm.at[c, pl.ds(s * DZR, DZR)])


def _deg_call(dst_r):
    mesh = plsc.VectorSubcoreMesh(core_axis_name="c", subcore_axis_name="s")
    return pl.kernel(
        _deg_body,
        out_type=jax.ShapeDtypeStruct((NC, DEG_PAD), jnp.float32),
        mesh=mesh,
        scratch_types=[
            pltpu.VMEM_SHARED((DEG_PAD,), jnp.float32),
            pltpu.VMEM((DCH, DK), jnp.int32),
            pltpu.VMEM((112,), jnp.float32),
            pltpu.VMEM((DZR,), jnp.float32),
        ],
    )(dst_r)


RB = 1000
NBLK = N // RB


def _prescale_body(parts_ref, x_ref, xsa_ref, xsb_ref):
    deg = jnp.sum(parts_ref[0], axis=0) + 1.0
    dinv = lax.rsqrt(deg)
    xs = x_ref[...] * dinv[:, None]
    xsa_ref[...] = xs[:, :128]
    xsb_ref[...] = xs[:, 128:]


def _prescale_call(deg_parts, x):
    return pl.pallas_call(
        _prescale_body,
        grid=(NBLK,),
        in_specs=[
            pl.BlockSpec((1, NC, RB), lambda i: (i, 0, 0)),
            pl.BlockSpec((RB, IN_F), lambda i: (i, 0)),
        ],
        out_specs=[
            pl.BlockSpec((RB, 128), lambda i: (i, 0)),
            pl.BlockSpec((RB, 128), lambda i: (i, 0)),
        ],
        out_shape=[
            jax.ShapeDtypeStruct((N, 128), jnp.float32),
            jax.ShapeDtypeStruct((N, 128), jnp.float32),
        ],
    )(deg_parts, x)


EK = 80
CPG = 25
NG = (E // NS) // (CPG * EK)
AN = 10240
RPT = AN // NS
ZR = 16


def _agg_body(src_hbm, dst_hbm, xsa_hbm, xsb_hbm, ax_hbm,
              acc, zbuf, sidx, didx, buf, sem):
    c = lax.axis_index("c")
    s = lax.axis_index("s")

    @pl.loop(0, ZR)
    def _zrow(r):
        for k in range(128 // L):
            zbuf[r, pl.ds(k * L, L)] = jnp.zeros((L,), jnp.float32)

    @pl.loop(0, RPT // ZR)
    def _zacc(b):
        pltpu.sync_copy(zbuf, acc.at[pl.ds(s * RPT + b * ZR, ZR)])

    plsc.subcore_barrier()

    def run(xs_hbm):
        @pl.loop(0, NG)
        def _grp(g):
            pltpu.sync_copy(src_hbm.at[s, g], sidx)
            pltpu.sync_copy(dst_hbm.at[s, g], didx)

            @pl.loop(0, CPG)
            def _edge(j):
                pltpu.async_copy(xs_hbm.at[sidx.at[j]], buf, sem).wait()
                pltpu.sync_copy(buf, acc.at[didx.at[j]], add=True)

    @pl.when(c == 0)
    def _():
        run(xsa_hbm)

    @pl.when(c == 1)
    def _():
        run(xsb_hbm)

    plsc.subcore_barrier()
    pltpu.sync_copy(acc.at[pl.ds(s * RPT, RPT)],
                    ax_hbm.at[c, pl.ds(s * RPT, RPT)])


def _agg_call(src_r, dst_r, xs_a, xs_b):
    mesh = plsc.VectorSubcoreMesh(core_axis_name="c", subcore_axis_name="s")
    return pl.kernel(
        _agg_body,
        out_type=jax.ShapeDtypeStruct((NC, AN, 128), jnp.float32),
        mesh=mesh,
        scratch_types=[
            pltpu.VMEM_SHARED((AN, 128), jnp.float32),
            pltpu.VMEM((ZR, 128), jnp.float32),
            pltpu.VMEM((CPG, EK), jnp.int32),
            pltpu.VMEM((CPG, EK), jnp.int32),
            pltpu.VMEM((EK, 128), jnp.float32),
            pltpu.SemaphoreType.DMA,
        ],
    )(src_r, dst_r, xs_a, xs_b)


def _tail_body(parts_ref, axa_ref, axb_ref, xsa_ref, xsb_ref,
               batch_blk_ref, batch_full_ref,
               W1_ref, b1_ref, W2_ref, b2_ref,
               node_ref, graph_ref, pool_acc):
    i = pl.program_id(0)
    deg = jnp.sum(parts_ref[0], axis=0) + 1.0
    dinv = lax.rsqrt(deg)
    agg_a = (axa_ref[...] + xsa_ref[...]) * dinv[:, None]
    agg_b = (axb_ref[...] + xsb_ref[...]) * dinv[:, None]
    agg = jnp.concatenate([agg_a, agg_b], axis=1)
    h = jnp.dot(agg, W1_ref[...], preferred_element_type=jnp.float32)
    h = jnp.tanh(h + b1_ref[...])
    node_ref[...] = h

    bblk = batch_blk_ref[0, 0, :]
    gids = lax.broadcasted_iota(jnp.int32, (G, RB), 0)
    P = (bblk[None, :] == gids).astype(jnp.float32)
    part = jnp.dot(P, h, preferred_element_type=jnp.float32)

    @pl.when(i == 0)
    def _():
        pool_acc[...] = part

    @pl.when(i > 0)
    def _():
        pool_acc[...] += part

    @pl.when(i == NBLK - 1)
    def _():
        bf = batch_full_ref[0, :]
        gall = lax.broadcasted_iota(jnp.int32, (G, N), 0)
        cnt = jnp.sum((bf[None, :] == gall).astype(jnp.float32), axis=1)
        mean = pool_acc[...] / jnp.maximum(cnt, 1.0)[:, None]
        g = jnp.dot(mean, W2_ref[...], preferred_element_type=jnp.float32)
        graph_ref[...] = jnp.tanh(g + b2_ref[...])


def _tail_call(deg_parts, ax_a, ax_b, xs_a, xs_b, batch, W1, b1, W2, b2):
    batch_blk = batch.reshape(NBLK, 1, RB)
    batch_full = batch.reshape(1, N)
    return pl.pallas_call(
        _tail_body,
        grid=(NBLK,),
        in_specs=[
            pl.BlockSpec((1, NC, RB), lambda i: (i, 0, 0)),
            pl.BlockSpec((RB, 128), lambda i: (i, 0)),
            pl.BlockSpec((RB, 128), lambda i: (i, 0)),
            pl.BlockSpec((RB, 128), lambda i: (i, 0)),
            pl.BlockSpec((RB, 128), lambda i: (i, 0)),
            pl.BlockSpec((1, 1, RB), lambda i: (i, 0, 0)),
            pl.BlockSpec((1, N), lambda i: (0, 0)),
            pl.BlockSpec((IN_F, HID_F), lambda i: (0, 0)),
            pl.BlockSpec((1, HID_F), lambda i: (0, 0)),
            pl.BlockSpec((HID_F, OUT_F), lambda i: (0, 0)),
            pl.BlockSpec((1, OUT_F), lambda i: (0, 0)),
        ],
        out_specs=[
            pl.BlockSpec((RB, HID_F), lambda i: (i, 0)),
            pl.BlockSpec((G, OUT_F), lambda i: (0, 0)),
        ],
        out_shape=[
            jax.ShapeDtypeStruct((N, HID_F), jnp.float32),
            jax.ShapeDtypeStruct((G, OUT_F), jnp.float32),
        ],
        scratch_shapes=[pltpu.VMEM((G, HID_F), jnp.float32)],
    )(deg_parts, ax_a, ax_b, xs_a, xs_b, batch_blk, batch_full,
      W1, b1.reshape(1, HID_F), W2, b2.reshape(1, OUT_F))


def kernel(x, edge_index, batch, W1, b1, W2, b2):
    src = edge_index[0].astype(jnp.int32)
    dst = edge_index[1].astype(jnp.int32)

    deg_parts = _deg_call(dst.reshape(NW, DCH, DK))
    deg_parts_t = deg_parts[:, :N].reshape(NC, NBLK, RB).transpose(1, 0, 2)

    xs_a, xs_b = _prescale_call(deg_parts_t, x)

    src_r = src.reshape(NS, NG, CPG, EK)
    dst_r = dst.reshape(NS, NG, CPG, EK)
    ax = _agg_call(src_r, dst_r, xs_a, xs_b)

    node, graph = _tail_call(deg_parts_t, ax[0, :N], ax[1, :N], xs_a, xs_b,
                             batch.astype(jnp.int32), W1, b1, W2, b2)
    return (graph, node)
